# baseline (speedup 1.0000x reference)
import functools

import jax
import jax.numpy as jnp
from jax import lax
from jax.experimental import pallas as pl
from jax.experimental.pallas import tpu as pltpu
from jax.experimental.pallas import tpu_sc as plsc

N_NODES = 10000
N_PAD = 10240
D = 128
N_CORES = 2
N_SUBCORES = 16
N_WORKERS = N_CORES * N_SUBCORES
CHUNK = 128
ROWS_PER_TILE = N_PAD // N_SUBCORES


def _ceil_to(x, m):
    return (x + m - 1) // m * m


def _spmm_sc(x, edata, wdata, m0):
    total_chunks = edata.shape[0]
    m1 = total_chunks // N_SUBCORES - m0
    assert m0 % 2 == 0 and m1 % 2 == 0
    mesh = plsc.VectorSubcoreMesh(core_axis_name="c", subcore_axis_name="s")

    @functools.partial(
        pl.kernel,
        out_type=jax.ShapeDtypeStruct((N_CORES, N_PAD, D), jnp.float32),
        mesh=mesh,
        scratch_types=[
            [pltpu.VMEM((2, CHUNK), jnp.int32) for _ in range(2)],
            [pltpu.VMEM((CHUNK,), jnp.float32) for _ in range(2)],
            [pltpu.VMEM((CHUNK, D), jnp.float32) for _ in range(2)],
            pltpu.VMEM_SHARED((N_PAD, D), jnp.float32),
            [pltpu.SemaphoreType.DMA for _ in range(2)],
            [pltpu.SemaphoreType.DMA for _ in range(2)],
        ],
    )
    def spmm_kernel(x_hbm, e_hbm, w_hbm, out_hbm, ebuf, wbuf, rows, acc, esem, gsem):
        c = lax.axis_index("c")
        s = lax.axis_index("s")
        n_chunks = jnp.where(c == 0, m0, m1)
        cbase = jnp.where(c == 0, s * m0, N_SUBCORES * m0 + s * m1)

        def zero_row(i, _):
            for j in range(D // 16):
                rows[0][i, pl.ds(j * 16, 16)] = jnp.zeros((16,), jnp.float32)
            return 0
        lax.fori_loop(0, CHUNK, zero_row, 0, unroll=False)
        for i in range(ROWS_PER_TILE // CHUNK):
            pltpu.sync_copy(
                rows[0], acc.at[pl.ds(s * ROWS_PER_TILE + i * CHUNK, CHUNK)])
        plsc.subcore_barrier()

        def fire_ecopy(k, b):
            pltpu.async_copy(e_hbm.at[cbase + k], ebuf[b], esem[b])
            pltpu.async_copy(w_hbm.at[cbase + k], wbuf[b], esem[b])

        def wait_ecopy(b):
            pltpu.make_async_copy(e_hbm.at[0], ebuf[b], esem[b]).wait()
            pltpu.make_async_copy(w_hbm.at[0], wbuf[b], esem[b]).wait()

        def fire_gather(b):
            pltpu.async_copy(x_hbm.at[ebuf[b].at[0]], rows[b], gsem[b])

        def wait_gather(b):
            pltpu.make_async_copy(x_hbm.at[ebuf[b].at[0]], rows[b],
                                  gsem[b]).wait()

        def scale_and_scatter(b):
            def group_body(g, _):
                w16 = wbuf[b][pl.ds(g * 16, 16)]
                for e in range(16):
                    wvec = jnp.full((16,), w16[e], jnp.float32)
                    abs_e = g * 16 + e
                    for j in range(D // 16):
                        rows[b][abs_e, pl.ds(j * 16, 16)] = (
                            rows[b][abs_e, pl.ds(j * 16, 16)] * wvec)
                return 0

            lax.fori_loop(0, CHUNK // 16, group_body, 0, unroll=False)
            pltpu.sync_copy(rows[b], acc.at[ebuf[b].at[1]], add=True)

        fire_ecopy(0, 0)
        wait_ecopy(0)
        fire_gather(0)
        fire_ecopy(1, 1)

        def pair_body(t, _):
            for b in range(2):
                k = t * 2 + b
                bo = 1 - b
                @pl.when(k + 1 < n_chunks)
                def _():
                    wait_ecopy(bo)
                    fire_gather(bo)
                wait_gather(b)
                scale_and_scatter(b)
                @pl.when(k + 2 < n_chunks)
                def _():
                    fire_ecopy(k + 2, b)
            return 0

        lax.fori_loop(0, lax.div(n_chunks, 2), pair_body, 0, unroll=False)
        plsc.subcore_barrier()
        pltpu.sync_copy(acc.at[pl.ds(s * ROWS_PER_TILE, ROWS_PER_TILE)],
                        out_hbm.at[c].at[pl.ds(s * ROWS_PER_TILE, ROWS_PER_TILE)])

    return spmm_kernel(x, edata, wdata)


_BLK = 1000


def _mm_body(x_ref, w_ref, o_ref):
    o_ref[...] = jnp.dot(x_ref[...], w_ref[...],
                         preferred_element_type=jnp.float32)


def _mm(x, w):
    n, d_in = x.shape
    d_out = w.shape[1]
    return pl.pallas_call(
        _mm_body,
        grid=(n // _BLK,),
        in_specs=[pl.BlockSpec((_BLK, d_in), lambda i: (i, 0)),
                  pl.BlockSpec((d_in, d_out), lambda i: (0, 0))],
        out_specs=pl.BlockSpec((_BLK, d_out), lambda i: (i, 0)),
        out_shape=jax.ShapeDtypeStruct((n, d_out), jnp.float32),
    )(x, w)


def _relu_mm_body(p0_ref, p1_ref, b_ref, w_ref, o_ref):
    h = jnp.maximum(p0_ref[...] + p1_ref[...] + b_ref[...], 0.0)
    o_ref[...] = jnp.dot(h, w_ref[...], preferred_element_type=jnp.float32)


def _relu_mm(p0, p1, b, w):
    n, d_in = p0.shape
    d_out = w.shape[1]
    return pl.pallas_call(
        _relu_mm_body,
        grid=(n // _BLK,),
        in_specs=[pl.BlockSpec((_BLK, d_in), lambda i: (i, 0)),
                  pl.BlockSpec((_BLK, d_in), lambda i: (i, 0)),
                  pl.BlockSpec((1, d_in), lambda i: (0, 0)),
                  pl.BlockSpec((d_in, d_out), lambda i: (0, 0))],
        out_specs=pl.BlockSpec((_BLK, d_out), lambda i: (i, 0)),
        out_shape=jax.ShapeDtypeStruct((n, d_out), jnp.float32),
    )(p0, p1, b.reshape(1, -1), w)


def _combine_body(q0_ref, q1_ref, b_ref, o_ref):
    o_ref[...] = q0_ref[...] + q1_ref[...] + b_ref[...]


def _combine(q0, q1, b):
    n, d = q0.shape
    return pl.pallas_call(
        _combine_body,
        grid=(n // _BLK,),
        in_specs=[pl.BlockSpec((_BLK, d), lambda i: (i, 0)),
                  pl.BlockSpec((_BLK, d), lambda i: (i, 0)),
                  pl.BlockSpec((1, d), lambda i: (0, 0))],
        out_specs=pl.BlockSpec((_BLK, d), lambda i: (i, 0)),
        out_shape=jax.ShapeDtypeStruct((n, d), jnp.float32),
    )(q0, q1, b.reshape(1, -1))


def kernel(features, edge_index, edge_weight, W1, b1, W2, b2, W3, b3):
    n_edges = edge_index.shape[1]
    e_pad = _ceil_to(n_edges, N_WORKERS * CHUNK * 2)
    total_chunks = e_pad // CHUNK

    m0 = int(total_chunks / N_SUBCORES * 0.275) // 2 * 2

    src = jnp.pad(edge_index[0].astype(jnp.int32), (0, e_pad - n_edges))
    dst = jnp.pad(edge_index[1].astype(jnp.int32), (0, e_pad - n_edges))
    w = jnp.pad(edge_weight.astype(jnp.float32), (0, e_pad - n_edges))
    edata = jnp.stack([src, dst]).reshape(2, total_chunks, CHUNK).transpose(1, 0, 2)
    wdata = w.reshape(total_chunks, CHUNK)

    xw1 = _mm(features, W1)
    p = _spmm_sc(xw1, edata, wdata, m0)

    W23 = jnp.concatenate([W2, W3], axis=1)
    hw = _relu_mm(p[0, :N_NODES], p[1, :N_NODES], b1, W23)
    q = _spmm_sc(hw, edata, wdata, m0)

    b23 = jnp.concatenate([b2, b3])
    out = _combine(q[0, :N_NODES], q[1, :N_NODES], b23)
    d_out = W2.shape[1]
    return out[:, :d_out], out[:, d_out:]


# speedup vs baseline: 1.1664x; 1.1664x over previous
import functools

import jax
import jax.numpy as jnp
from jax import lax
from jax.experimental import pallas as pl
from jax.experimental.pallas import tpu as pltpu
from jax.experimental.pallas import tpu_sc as plsc

N_NODES = 10000
N_PAD = 10240
D = 128
N_CORES = 2
N_SUBCORES = 16
N_WORKERS = N_CORES * N_SUBCORES
CHUNK = 128
ROWS_PER_TILE = N_PAD // N_SUBCORES


def _ceil_to(x, m):
    return (x + m - 1) // m * m


def _spmm_sc(x, edata, wdata, m0):
    total_chunks = edata.shape[0]
    m1 = total_chunks // N_SUBCORES - m0
    assert m0 % 2 == 0 and m1 % 2 == 0
    mesh = plsc.VectorSubcoreMesh(core_axis_name="c", subcore_axis_name="s")

    @functools.partial(
        pl.kernel,
        out_type=jax.ShapeDtypeStruct((N_CORES, N_PAD, D), jnp.float32),
        mesh=mesh,
        scratch_types=[
            [pltpu.VMEM((2, CHUNK), jnp.int32) for _ in range(2)],
            [pltpu.VMEM((CHUNK,), jnp.float32) for _ in range(2)],
            [pltpu.VMEM((CHUNK, D), jnp.float32) for _ in range(2)],
            pltpu.VMEM_SHARED((N_PAD, D), jnp.float32),
            [pltpu.SemaphoreType.DMA for _ in range(2)],
            [pltpu.SemaphoreType.DMA for _ in range(2)],
        ],
    )
    def spmm_kernel(x_hbm, e_hbm, w_hbm, out_hbm, ebuf, wbuf, rows, acc, esem, gsem):
        c = lax.axis_index("c")
        s = lax.axis_index("s")
        n_chunks = jnp.where(c == 0, m0, m1)
        cbase = jnp.where(c == 0, s * m0, N_SUBCORES * m0 + s * m1)

        def zero_row(i, _):
            for j in range(D // 16):
                rows[0][i, pl.ds(j * 16, 16)] = jnp.zeros((16,), jnp.float32)
            return 0
        lax.fori_loop(0, CHUNK, zero_row, 0, unroll=False)
        for i in range(ROWS_PER_TILE // CHUNK):
            pltpu.sync_copy(
                rows[0], acc.at[pl.ds(s * ROWS_PER_TILE + i * CHUNK, CHUNK)])
        plsc.subcore_barrier()

        def fire_ecopy(k, b):
            pltpu.async_copy(e_hbm.at[cbase + k], ebuf[b], esem[b])
            pltpu.async_copy(w_hbm.at[cbase + k], wbuf[b], esem[b])

        def wait_ecopy(b):
            pltpu.make_async_copy(e_hbm.at[0], ebuf[b], esem[b]).wait()
            pltpu.make_async_copy(w_hbm.at[0], wbuf[b], esem[b]).wait()

        def fire_gather(b):
            pltpu.async_copy(x_hbm.at[ebuf[b].at[0]], rows[b], gsem[b])

        def wait_gather(b):
            pltpu.make_async_copy(x_hbm.at[ebuf[b].at[0]], rows[b],
                                  gsem[b]).wait()

        def scale_and_scatter(b):
            def group_body(g, _):
                w16 = wbuf[b][pl.ds(g * 16, 16)]
                for e in range(16):
                    wvec = jnp.full((16,), w16[e], jnp.float32)
                    abs_e = g * 16 + e
                    for j in range(D // 16):
                        rows[b][abs_e, pl.ds(j * 16, 16)] = (
                            rows[b][abs_e, pl.ds(j * 16, 16)] * wvec)
                return 0

            lax.fori_loop(0, CHUNK // 16, group_body, 0, unroll=False)
            pltpu.sync_copy(rows[b], acc.at[ebuf[b].at[1]], add=True)

        fire_ecopy(0, 0)
        wait_ecopy(0)
        fire_gather(0)
        fire_ecopy(1, 1)

        def pair_body(t, _):
            for b in range(2):
                k = t * 2 + b
                bo = 1 - b
                @pl.when(k + 1 < n_chunks)
                def _():
                    wait_ecopy(bo)
                    fire_gather(bo)
                wait_gather(b)
                scale_and_scatter(b)
                @pl.when(k + 2 < n_chunks)
                def _():
                    fire_ecopy(k + 2, b)
            return 0

        lax.fori_loop(0, lax.div(n_chunks, 2), pair_body, 0, unroll=False)
        plsc.subcore_barrier()
        pltpu.sync_copy(acc.at[pl.ds(s * ROWS_PER_TILE, ROWS_PER_TILE)],
                        out_hbm.at[c].at[pl.ds(s * ROWS_PER_TILE, ROWS_PER_TILE)])

    return spmm_kernel(x, edata, wdata)


_BLK = 1000


def _mm_body(x_ref, w_ref, o_ref):
    o_ref[...] = jnp.dot(x_ref[...], w_ref[...],
                         preferred_element_type=jnp.float32)


def _mm(x, w):
    n, d_in = x.shape
    d_out = w.shape[1]
    return pl.pallas_call(
        _mm_body,
        grid=(n // _BLK,),
        in_specs=[pl.BlockSpec((_BLK, d_in), lambda i: (i, 0)),
                  pl.BlockSpec((d_in, d_out), lambda i: (0, 0))],
        out_specs=pl.BlockSpec((_BLK, d_out), lambda i: (i, 0)),
        out_shape=jax.ShapeDtypeStruct((n, d_out), jnp.float32),
    )(x, w)


def _relu_mm_body(p0_ref, p1_ref, b_ref, w_ref, o_ref):
    h = jnp.maximum(p0_ref[...] + p1_ref[...] + b_ref[...], 0.0)
    o_ref[...] = jnp.dot(h, w_ref[...], preferred_element_type=jnp.float32)


def _relu_mm(p0, p1, b, w):
    n, d_in = p0.shape
    d_out = w.shape[1]
    return pl.pallas_call(
        _relu_mm_body,
        grid=(n // _BLK,),
        in_specs=[pl.BlockSpec((_BLK, d_in), lambda i: (i, 0)),
                  pl.BlockSpec((_BLK, d_in), lambda i: (i, 0)),
                  pl.BlockSpec((1, d_in), lambda i: (0, 0)),
                  pl.BlockSpec((d_in, d_out), lambda i: (0, 0))],
        out_specs=pl.BlockSpec((_BLK, d_out), lambda i: (i, 0)),
        out_shape=jax.ShapeDtypeStruct((n, d_out), jnp.float32),
    )(p0, p1, b.reshape(1, -1), w)


def _combine_body(q0_ref, q1_ref, b_ref, o_ref):
    o_ref[...] = q0_ref[...] + q1_ref[...] + b_ref[...]


def _combine(q0, q1, b):
    n, d = q0.shape
    return pl.pallas_call(
        _combine_body,
        grid=(n // _BLK,),
        in_specs=[pl.BlockSpec((_BLK, d), lambda i: (i, 0)),
                  pl.BlockSpec((_BLK, d), lambda i: (i, 0)),
                  pl.BlockSpec((1, d), lambda i: (0, 0))],
        out_specs=pl.BlockSpec((_BLK, d), lambda i: (i, 0)),
        out_shape=jax.ShapeDtypeStruct((n, d), jnp.float32),
    )(q0, q1, b.reshape(1, -1))


def kernel(features, edge_index, edge_weight, W1, b1, W2, b2, W3, b3):
    n_edges = edge_index.shape[1]
    e_pad = _ceil_to(n_edges, N_WORKERS * CHUNK * 2)
    total_chunks = e_pad // CHUNK

    m0 = int(total_chunks / N_SUBCORES * 0.725) // 2 * 2

    src = jnp.pad(edge_index[0].astype(jnp.int32), (0, e_pad - n_edges))
    dst = jnp.pad(edge_index[1].astype(jnp.int32), (0, e_pad - n_edges))
    w = jnp.pad(edge_weight.astype(jnp.float32), (0, e_pad - n_edges))
    edata = jnp.stack([src, dst]).reshape(2, total_chunks, CHUNK).transpose(1, 0, 2)
    wdata = w.reshape(total_chunks, CHUNK)

    xw1 = _mm(features, W1)
    p = _spmm_sc(xw1, edata, wdata, m0)

    W23 = jnp.concatenate([W2, W3], axis=1)
    hw = _relu_mm(p[0, :N_NODES], p[1, :N_NODES], b1, W23)
    q = _spmm_sc(hw, edata, wdata, m0)

    b23 = jnp.concatenate([b2, b3])
    out = _combine(q[0, :N_NODES], q[1, :N_NODES], b23)
    d_out = W2.shape[1]
    return out[:, :d_out], out[:, d_out:]


# speedup vs baseline: 1.3263x; 1.1371x over previous
import functools

import jax
import jax.numpy as jnp
import numpy as np
from jax import lax
from jax.experimental import pallas as pl
from jax.experimental.pallas import tpu as pltpu
from jax.experimental.pallas import tpu_sc as plsc

N_NODES = 10000
N_PAD = 10240
D = 128
N_CORES = 2
N_SUBCORES = 16
N_WORKERS = N_CORES * N_SUBCORES
CHUNK = 128
ROWS_PER_TILE = N_PAD // N_SUBCORES


def _ceil_to(x, m):
    return (x + m - 1) // m * m


def _spmm_sc(x, edata, wdata, m0):
    total_chunks = edata.shape[0]
    m1 = total_chunks // N_SUBCORES - m0
    assert m0 % 2 == 0 and m1 % 2 == 0
    mesh = plsc.VectorSubcoreMesh(core_axis_name="c", subcore_axis_name="s")

    @functools.partial(
        pl.kernel,
        out_type=jax.ShapeDtypeStruct((N_CORES, N_PAD, D), jnp.float32),
        mesh=mesh,
        scratch_types=[
            [pltpu.VMEM((2, CHUNK), jnp.int32) for _ in range(2)],
            [pltpu.VMEM((CHUNK,), jnp.float32) for _ in range(2)],
            [pltpu.VMEM((CHUNK, D // 2), jnp.int32) for _ in range(2)],
            pltpu.VMEM((CHUNK, D), jnp.float32),
            pltpu.VMEM_SHARED((N_PAD, D), jnp.float32),
            [pltpu.SemaphoreType.DMA for _ in range(2)],
            [pltpu.SemaphoreType.DMA for _ in range(2)],
        ],
        compiler_params=pltpu.CompilerParams(needs_layout_passes=False,
                                             use_tc_tiling_on_sc=False),
    )
    def spmm_kernel(x_hbm, e_hbm, w_hbm, out_hbm, ebuf, wbuf, rows, rowsf, acc, esem, gsem):
        c = lax.axis_index("c")
        s = lax.axis_index("s")
        n_chunks = jnp.where(c == 0, m0, m1)
        cbase = jnp.where(c == 0, s * m0, N_SUBCORES * m0 + s * m1)

        def zero_row(i, _):
            for j in range(D // 16):
                rowsf[i, pl.ds(j * 16, 16)] = jnp.zeros((16,), jnp.float32)
            return 0
        lax.fori_loop(0, CHUNK, zero_row, 0, unroll=False)
        for i in range(ROWS_PER_TILE // CHUNK):
            pltpu.sync_copy(
                rowsf, acc.at[pl.ds(s * ROWS_PER_TILE + i * CHUNK, CHUNK)])
        plsc.subcore_barrier()

        def fire_ecopy(k, b):
            pltpu.async_copy(e_hbm.at[cbase + k], ebuf[b], esem[b])
            pltpu.async_copy(w_hbm.at[cbase + k], wbuf[b], esem[b])

        def wait_ecopy(b):
            pltpu.make_async_copy(e_hbm.at[0], ebuf[b], esem[b]).wait()
            pltpu.make_async_copy(w_hbm.at[0], wbuf[b], esem[b]).wait()

        def fire_gather(b):
            pltpu.async_copy(x_hbm.at[ebuf[b].at[0]], rows[b], gsem[b])

        def wait_gather(b):
            pltpu.make_async_copy(x_hbm.at[ebuf[b].at[0]], rows[b],
                                  gsem[b]).wait()

        def scale_and_scatter(b):
            def group_body(g, _):
                w16 = wbuf[b][pl.ds(g * 16, 16)]
                for e in range(16):
                    wvec = jnp.full((16,), w16[e], jnp.float32)
                    abs_e = g * 16 + e
                    for h in range(D // 32):
                        v = plsc.bitcast(rows[b][abs_e, pl.ds(h * 16, 16)],
                                         jnp.bfloat16)
                        lo, hi = plsc.unpack(
                            v, format=plsc.PackFormat.INTERLEAVED)
                        rowsf[abs_e, pl.ds(h * 32, 16)] = lo * wvec
                        rowsf[abs_e, pl.ds(h * 32 + 16, 16)] = hi * wvec
                return 0

            lax.fori_loop(0, CHUNK // 16, group_body, 0, unroll=False)
            pltpu.sync_copy(rowsf, acc.at[ebuf[b].at[1]], add=True)

        fire_ecopy(0, 0)
        wait_ecopy(0)
        fire_gather(0)
        fire_ecopy(1, 1)

        def pair_body(t, _):
            for b in range(2):
                k = t * 2 + b
                bo = 1 - b
                @pl.when(k + 1 < n_chunks)
                def _():
                    wait_ecopy(bo)
                    fire_gather(bo)
                wait_gather(b)
                scale_and_scatter(b)
                @pl.when(k + 2 < n_chunks)
                def _():
                    fire_ecopy(k + 2, b)
            return 0

        lax.fori_loop(0, lax.div(n_chunks, 2), pair_body, 0, unroll=False)
        plsc.subcore_barrier()
        pltpu.sync_copy(acc.at[pl.ds(s * ROWS_PER_TILE, ROWS_PER_TILE)],
                        out_hbm.at[c].at[pl.ds(s * ROWS_PER_TILE, ROWS_PER_TILE)])

    return spmm_kernel(x, edata, wdata)


_BLK = 1000


def _mm_body(x_ref, w_ref, o_ref):
    o_ref[...] = jnp.dot(x_ref[...], w_ref[...],
                         preferred_element_type=jnp.float32).astype(jnp.bfloat16)


def _mm(x, w):
    n, d_in = x.shape
    d_out = w.shape[1]
    return pl.pallas_call(
        _mm_body,
        grid=(n // _BLK,),
        in_specs=[pl.BlockSpec((_BLK, d_in), lambda i: (i, 0)),
                  pl.BlockSpec((d_in, d_out), lambda i: (0, 0))],
        out_specs=pl.BlockSpec((_BLK, d_out), lambda i: (i, 0)),
        out_shape=jax.ShapeDtypeStruct((n, d_out), jnp.bfloat16),
    )(x, w)


def _relu_mm_body(p0_ref, p1_ref, b_ref, w_ref, o_ref):
    h = jnp.maximum(p0_ref[...] + p1_ref[...] + b_ref[...], 0.0)
    o_ref[...] = jnp.dot(h, w_ref[...],
                         preferred_element_type=jnp.float32).astype(jnp.bfloat16)


def _relu_mm(p0, p1, b, w):
    n, d_in = p0.shape
    d_out = w.shape[1]
    return pl.pallas_call(
        _relu_mm_body,
        grid=(n // _BLK,),
        in_specs=[pl.BlockSpec((_BLK, d_in), lambda i: (i, 0)),
                  pl.BlockSpec((_BLK, d_in), lambda i: (i, 0)),
                  pl.BlockSpec((1, d_in), lambda i: (0, 0)),
                  pl.BlockSpec((d_in, d_out), lambda i: (0, 0))],
        out_specs=pl.BlockSpec((_BLK, d_out), lambda i: (i, 0)),
        out_shape=jax.ShapeDtypeStruct((n, d_out), jnp.bfloat16),
    )(p0, p1, b.reshape(1, -1), w)


def _combine_body(q0_ref, q1_ref, b_ref, o_ref):
    o_ref[...] = q0_ref[...] + q1_ref[...] + b_ref[...]


def _combine(q0, q1, b):
    n, d = q0.shape
    return pl.pallas_call(
        _combine_body,
        grid=(n // _BLK,),
        in_specs=[pl.BlockSpec((_BLK, d), lambda i: (i, 0)),
                  pl.BlockSpec((_BLK, d), lambda i: (i, 0)),
                  pl.BlockSpec((1, d), lambda i: (0, 0))],
        out_specs=pl.BlockSpec((_BLK, d), lambda i: (i, 0)),
        out_shape=jax.ShapeDtypeStruct((n, d), jnp.float32),
    )(q0, q1, b.reshape(1, -1))


def kernel(features, edge_index, edge_weight, W1, b1, W2, b2, W3, b3):
    n_edges = edge_index.shape[1]
    e_pad = _ceil_to(n_edges, N_WORKERS * CHUNK * 2)
    total_chunks = e_pad // CHUNK

    m0 = total_chunks // N_CORES // N_SUBCORES // 2 * 2

    src = jnp.pad(edge_index[0].astype(jnp.int32), (0, e_pad - n_edges))
    dst = jnp.pad(edge_index[1].astype(jnp.int32), (0, e_pad - n_edges))
    w = jnp.pad(edge_weight.astype(jnp.float32), (0, e_pad - n_edges))
    edata = jnp.stack([src, dst]).reshape(2, total_chunks, CHUNK).transpose(1, 0, 2)
    wdata = w.reshape(total_chunks, CHUNK)

    perm = np.arange(D).reshape(D // 32, 16, 2).transpose(0, 2, 1).reshape(-1)
    inv_perm = np.argsort(perm)

    def _as_i32(a_bf16):
        n = a_bf16.shape[0]
        return jax.lax.bitcast_convert_type(
            a_bf16.reshape(n, D // 2, 2), jnp.int32)

    xw1 = _mm(features, W1)
    p = _spmm_sc(_as_i32(xw1), edata, wdata, m0)

    W23 = jnp.concatenate([W2, W3], axis=1)
    hw = _relu_mm(p[0, :N_NODES], p[1, :N_NODES], b1[perm], W23[perm, :])
    q = _spmm_sc(_as_i32(hw), edata, wdata, m0)

    b23 = jnp.concatenate([b2, b3])
    out = _combine(q[0, :N_NODES], q[1, :N_NODES], b23[perm])
    out = out[:, inv_perm]
    d_out = W2.shape[1]
    return out[:, :d_out], out[:, d_out:]


# speedup vs baseline: 1.3674x; 1.0309x over previous
import functools

import jax
import jax.numpy as jnp
import numpy as np
from jax import lax
from jax.experimental import pallas as pl
from jax.experimental.pallas import tpu as pltpu
from jax.experimental.pallas import tpu_sc as plsc

N_NODES = 10000
N_PAD = 10240
D = 128
N_CORES = 2
N_SUBCORES = 16
N_WORKERS = N_CORES * N_SUBCORES
CHUNK = 112
ROWS_PER_TILE = N_PAD // N_SUBCORES


def _ceil_to(x, m):
    return (x + m - 1) // m * m


def _spmm_sc(x, edata, m0):
    total_chunks = edata.shape[0]
    m1 = total_chunks // N_SUBCORES - m0
    assert m0 % 4 == 0 and m1 % 4 == 0
    mesh = plsc.VectorSubcoreMesh(core_axis_name="c", subcore_axis_name="s")

    @functools.partial(
        pl.kernel,
        out_type=jax.ShapeDtypeStruct((N_CORES, N_PAD, D), jnp.float32),
        mesh=mesh,
        scratch_types=[
            [pltpu.VMEM((3, CHUNK), jnp.int32) for _ in range(4)],
            [pltpu.VMEM((CHUNK, D // 2), jnp.int32) for _ in range(2)],
            [pltpu.VMEM((CHUNK, D), jnp.float32) for _ in range(2)],
            pltpu.VMEM_SHARED((N_PAD, D), jnp.float32),
            [pltpu.SemaphoreType.DMA for _ in range(4)],
            [pltpu.SemaphoreType.DMA for _ in range(2)],
            [pltpu.SemaphoreType.DMA for _ in range(2)],
        ],
        compiler_params=pltpu.CompilerParams(needs_layout_passes=False,
                                             use_tc_tiling_on_sc=False,
                                             internal_scratch_in_bytes=0),
    )
    def spmm_kernel(x_hbm, e_hbm, out_hbm, ebuf, rows, rowsf,
                    acc, esem, gsem, ssem):
        c = lax.axis_index("c")
        s = lax.axis_index("s")
        n_chunks = jnp.where(c == 0, m0, m1)
        cbase = jnp.where(c == 0, s * m0, N_SUBCORES * m0 + s * m1)

        def zero_row(i, _):
            for j in range(D // 16):
                rowsf[0][i, pl.ds(j * 16, 16)] = jnp.zeros((16,), jnp.float32)
            return 0
        lax.fori_loop(0, CHUNK, zero_row, 0, unroll=False)
        off = 0
        while off < ROWS_PER_TILE:
            ln = min(CHUNK, ROWS_PER_TILE - off)
            pltpu.sync_copy(
                rowsf[0].at[pl.ds(0, ln)],
                acc.at[pl.ds(s * ROWS_PER_TILE + off, ln)])
            off += ln
        plsc.subcore_barrier()

        def fire_ecopy(k, eb):
            pltpu.async_copy(e_hbm.at[cbase + k], ebuf[eb], esem[eb])

        def wait_ecopy(eb):
            pltpu.make_async_copy(e_hbm.at[0], ebuf[eb], esem[eb]).wait()

        def fire_gather(eb, rb):
            pltpu.async_copy(x_hbm.at[ebuf[eb].at[0]], rows[rb], gsem[rb])

        def wait_gather(eb, rb):
            pltpu.make_async_copy(x_hbm.at[ebuf[eb].at[0]], rows[rb],
                                  gsem[rb]).wait()

        def scale(eb, rb):
            def group_body(g, _):
                w16 = plsc.bitcast(ebuf[eb][2, pl.ds(g * 16, 16)],
                                   jnp.float32)
                for e in range(16):
                    wvec = jnp.full((16,), w16[e], jnp.float32)
                    abs_e = g * 16 + e
                    for h in range(D // 32):
                        v = plsc.bitcast(rows[rb][abs_e, pl.ds(h * 16, 16)],
                                         jnp.bfloat16)
                        lo, hi = plsc.unpack(
                            v, format=plsc.PackFormat.INTERLEAVED)
                        rowsf[rb][abs_e, pl.ds(h * 32, 16)] = lo * wvec
                        rowsf[rb][abs_e, pl.ds(h * 32 + 16, 16)] = hi * wvec
                return 0

            lax.fori_loop(0, CHUNK // 16, group_body, 0, unroll=False)

        def drain_scatter(rb):
            pltpu.make_async_copy(rowsf[rb], acc.at[ebuf[0].at[1]],
                                  ssem[rb]).wait()

        fire_ecopy(0, 0)
        wait_ecopy(0)
        fire_gather(0, 0)
        fire_ecopy(1, 1)

        def quad_body(t, _):
            for eb in range(4):
                k = t * 4 + eb
                rb = eb % 2
                @pl.when(k + 1 < n_chunks)
                def _():
                    wait_ecopy((eb + 1) % 4)
                    fire_gather((eb + 1) % 4, (rb + 1) % 2)
                wait_gather(eb, rb)
                @pl.when(k >= 2)
                def _():
                    drain_scatter(rb)
                scale(eb, rb)
                pltpu.async_copy(rowsf[rb], acc.at[ebuf[eb].at[1]], ssem[rb],
                                 add=True)
                @pl.when(k + 2 < n_chunks)
                def _():
                    fire_ecopy(k + 2, (eb + 2) % 4)
            return 0

        lax.fori_loop(0, lax.div(n_chunks, 4), quad_body, 0, unroll=False)
        for rb in range(2):
            drain_scatter(rb)
        plsc.subcore_barrier()
        pltpu.sync_copy(acc.at[pl.ds(s * ROWS_PER_TILE, ROWS_PER_TILE)],
                        out_hbm.at[c].at[pl.ds(s * ROWS_PER_TILE, ROWS_PER_TILE)])

    return spmm_kernel(x, edata)


_BLK = 1000


def _mm_body(x_ref, w_ref, o_ref):
    o_ref[...] = jnp.dot(x_ref[...], w_ref[...],
                         preferred_element_type=jnp.float32).astype(jnp.bfloat16)


def _mm(x, w):
    n, d_in = x.shape
    d_out = w.shape[1]
    return pl.pallas_call(
        _mm_body,
        grid=(n // _BLK,),
        in_specs=[pl.BlockSpec((_BLK, d_in), lambda i: (i, 0)),
                  pl.BlockSpec((d_in, d_out), lambda i: (0, 0))],
        out_specs=pl.BlockSpec((_BLK, d_out), lambda i: (i, 0)),
        out_shape=jax.ShapeDtypeStruct((n, d_out), jnp.bfloat16),
    )(x, w)


def _relu_mm_body(p0_ref, p1_ref, b_ref, w_ref, o_ref):
    h = jnp.maximum(p0_ref[...] + p1_ref[...] + b_ref[...], 0.0)
    o_ref[...] = jnp.dot(h, w_ref[...],
                         preferred_element_type=jnp.float32).astype(jnp.bfloat16)


def _relu_mm(p0, p1, b, w):
    n, d_in = p0.shape
    d_out = w.shape[1]
    return pl.pallas_call(
        _relu_mm_body,
        grid=(n // _BLK,),
        in_specs=[pl.BlockSpec((_BLK, d_in), lambda i: (i, 0)),
                  pl.BlockSpec((_BLK, d_in), lambda i: (i, 0)),
                  pl.BlockSpec((1, d_in), lambda i: (0, 0)),
                  pl.BlockSpec((d_in, d_out), lambda i: (0, 0))],
        out_specs=pl.BlockSpec((_BLK, d_out), lambda i: (i, 0)),
        out_shape=jax.ShapeDtypeStruct((n, d_out), jnp.bfloat16),
    )(p0, p1, b.reshape(1, -1), w)


def _combine_body(q0_ref, q1_ref, b_ref, o_ref):
    o_ref[...] = q0_ref[...] + q1_ref[...] + b_ref[...]


def _combine(q0, q1, b):
    n, d = q0.shape
    return pl.pallas_call(
        _combine_body,
        grid=(n // _BLK,),
        in_specs=[pl.BlockSpec((_BLK, d), lambda i: (i, 0)),
                  pl.BlockSpec((_BLK, d), lambda i: (i, 0)),
                  pl.BlockSpec((1, d), lambda i: (0, 0))],
        out_specs=pl.BlockSpec((_BLK, d), lambda i: (i, 0)),
        out_shape=jax.ShapeDtypeStruct((n, d), jnp.float32),
    )(q0, q1, b.reshape(1, -1))


def kernel(features, edge_index, edge_weight, W1, b1, W2, b2, W3, b3):
    n_edges = edge_index.shape[1]
    e_pad = _ceil_to(n_edges, N_WORKERS * CHUNK * 4)
    total_chunks = e_pad // CHUNK

    m0 = total_chunks // N_CORES // N_SUBCORES // 4 * 4

    src = jnp.pad(edge_index[0].astype(jnp.int32), (0, e_pad - n_edges))
    dst = jnp.pad(edge_index[1].astype(jnp.int32), (0, e_pad - n_edges))
    w = jnp.pad(edge_weight.astype(jnp.float32), (0, e_pad - n_edges))
    edata = jnp.stack([src, dst, jax.lax.bitcast_convert_type(w, jnp.int32)])
    edata = edata.reshape(3, total_chunks, CHUNK).transpose(1, 0, 2)

    perm = np.arange(D).reshape(D // 32, 16, 2).transpose(0, 2, 1).reshape(-1)
    inv_perm = np.argsort(perm)

    def _as_i32(a_bf16):
        n = a_bf16.shape[0]
        return jax.lax.bitcast_convert_type(
            a_bf16.reshape(n, D // 2, 2), jnp.int32)

    xw1 = _mm(features, W1)
    p = _spmm_sc(_as_i32(xw1), edata, m0)

    W23 = jnp.concatenate([W2, W3], axis=1)
    hw = _relu_mm(p[0, :N_NODES], p[1, :N_NODES], b1[perm], W23[perm, :])
    q = _spmm_sc(_as_i32(hw), edata, m0)

    b23 = jnp.concatenate([b2, b3])
    out = _combine(q[0, :N_NODES], q[1, :N_NODES], b23[perm])
    out = out[:, inv_perm]
    d_out = W2.shape[1]
    return out[:, :d_out], out[:, d_out:]
